# Initial kernel scaffold; baseline (speedup 1.0000x reference)
import jax
import jax.numpy as jnp
from jax.experimental import pallas as pl


def kernel(x, c1_w, c1_b, c2_w, c2_b, c3_w, c3_b, f1_w, f1_b, f2_w, f2_b):
    raise NotImplementedError("write your pallas kernel here")


# speedup vs baseline: 3.1046x; 3.1046x over previous
import functools

import jax
import jax.numpy as jnp
from jax.experimental import pallas as pl
from jax.experimental.pallas import tpu as pltpu

_BF = jnp.bfloat16


def _conv_tail(y, b_ref, scr_ref, *, l_in, pool):
    m = y.shape[0]
    y_prev = y[:, 0:128]
    y_mid = y[:, 128:256]
    y_next = y[:, 256:384]
    pos = jax.lax.broadcasted_iota(jnp.int32, (m, 128), 0) % l_in
    conv = y_mid
    conv = conv + jnp.where(pos == 0, 0.0, jnp.roll(y_prev, 1, axis=0))
    conv = conv + jnp.where(pos == l_in - 1, 0.0, jnp.roll(y_next, -1, axis=0))
    if pool:
        scr_ref[...] = conv
        even = scr_ref[pl.ds(0, m // 2, 2), :]
        odd = scr_ref[pl.ds(1, m // 2, 2), :]
        z = jnp.maximum(even, odd)
    else:
        z = conv
    return jnp.maximum(z + b_ref[...], 0.0).astype(_BF)


def _fused_convs_kernel(x_ref, w1_ref, b1_ref, w2_ref, b2_ref, w3_ref, b3_ref,
                        o_ref, scr1, scr2, *, bblk, seq):
    w1 = w1_ref[...].astype(_BF)
    parts = []
    for b in range(bblk):
        xb = x_ref[b].astype(_BF)
        parts.append(jax.lax.dot_general(
            xb, w1, (((0,), (0,)), ((), ())),
            preferred_element_type=jnp.float32))
    y1 = jnp.concatenate(parts, axis=0) if bblk > 1 else parts[0]
    a1 = _conv_tail(y1, b1_ref, scr1, l_in=seq, pool=True)

    y2 = jnp.dot(a1, w2_ref[...].astype(_BF),
                 preferred_element_type=jnp.float32)
    a2 = _conv_tail(y2, b2_ref, scr2, l_in=seq // 2, pool=True)

    y3 = jnp.dot(a2, w3_ref[...].astype(_BF),
                 preferred_element_type=jnp.float32)
    a3 = _conv_tail(y3, b3_ref, None, l_in=seq // 4, pool=False)

    o_ref[...] = a3


def _fc_head_kernel(a_ref, w1_ref, b1_ref, w2_ref, b2_ref, o_ref, acc_ref):
    @pl.when(pl.program_id(1) == 0)
    def _():
        acc_ref[...] = jnp.zeros_like(acc_ref)

    acc_ref[...] += jnp.dot(a_ref[...], w1_ref[...].astype(_BF),
                            preferred_element_type=jnp.float32)

    @pl.when(pl.program_id(1) == pl.num_programs(1) - 1)
    def _():
        h = jnp.maximum(acc_ref[...] + b1_ref[...], 0.0).astype(_BF)
        out = jnp.dot(h, w2_ref[...].astype(_BF),
                      preferred_element_type=jnp.float32)
        o_ref[...] = out + b2_ref[...]


def kernel(x, c1_w, c1_b, c2_w, c2_b, c3_w, c3_b, f1_w, f1_b, f2_w, f2_b):
    batch, cin, seq = x.shape
    l4 = seq // 4
    bblk = next(d for d in (4, 2, 1) if batch % d == 0)
    m_out = bblk * l4

    body = functools.partial(_fused_convs_kernel, bblk=bblk, seq=seq)
    act = pl.pallas_call(
        body,
        out_shape=jax.ShapeDtypeStruct((batch * l4, 128), _BF),
        grid=(batch // bblk,),
        in_specs=[
            pl.BlockSpec((bblk, cin, seq), lambda i: (i, 0, 0)),
            pl.BlockSpec((cin, 384), lambda i: (0, 0)),
            pl.BlockSpec((1, 128), lambda i: (0, 0)),
            pl.BlockSpec((128, 384), lambda i: (0, 0)),
            pl.BlockSpec((1, 128), lambda i: (0, 0)),
            pl.BlockSpec((128, 384), lambda i: (0, 0)),
            pl.BlockSpec((1, 128), lambda i: (0, 0)),
        ],
        out_specs=pl.BlockSpec((m_out, 128), lambda i: (i, 0)),
        scratch_shapes=[
            pltpu.VMEM((bblk * seq, 128), jnp.float32),
            pltpu.VMEM((bblk * (seq // 2), 128), jnp.float32),
        ],
        compiler_params=pltpu.CompilerParams(
            dimension_semantics=("parallel",)),
    )(x, c1_w, c1_b, c2_w, c2_b, c3_w, c3_b)

    a = act.reshape(batch, l4 * 128)
    k_tot = l4 * 128
    tk = min(8192, k_tot)
    bm = batch // 2 if batch % 2 == 0 else batch
    out = pl.pallas_call(
        _fc_head_kernel,
        out_shape=jax.ShapeDtypeStruct((batch, 128), jnp.float32),
        grid=(batch // bm, k_tot // tk),
        in_specs=[
            pl.BlockSpec((bm, tk), lambda i, ki: (i, ki)),
            pl.BlockSpec((tk, 128), lambda i, ki: (ki, 0)),
            pl.BlockSpec((1, 128), lambda i, ki: (0, 0)),
            pl.BlockSpec((128, 128), lambda i, ki: (0, 0)),
            pl.BlockSpec((1, 128), lambda i, ki: (0, 0)),
        ],
        out_specs=pl.BlockSpec((bm, 128), lambda i, ki: (i, 0)),
        scratch_shapes=[pltpu.VMEM((bm, 128), jnp.float32)],
        compiler_params=pltpu.CompilerParams(
            dimension_semantics=("parallel", "arbitrary")),
    )(a, f1_w, f1_b, f2_w, f2_b)
    return out[:, :10]


# speedup vs baseline: 3.4885x; 1.1237x over previous
import functools

import jax
import jax.numpy as jnp
from jax.experimental import pallas as pl
from jax.experimental.pallas import tpu as pltpu

_BF = jnp.bfloat16


def _conv_tail(y, b_ref, scr_ref, *, l_in, pool):
    m = y.shape[0]
    y_prev = y[:, 0:128]
    y_mid = y[:, 128:256]
    y_next = y[:, 256:384]
    pos = jax.lax.broadcasted_iota(jnp.int32, (m, 128), 0) % l_in
    conv = y_mid
    conv = conv + jnp.where(pos == 0, 0.0, jnp.roll(y_prev, 1, axis=0))
    conv = conv + jnp.where(pos == l_in - 1, 0.0, jnp.roll(y_next, -1, axis=0))
    if pool:
        scr_ref[...] = conv
        even = scr_ref[pl.ds(0, m // 2, 2), :]
        odd = scr_ref[pl.ds(1, m // 2, 2), :]
        z = jnp.maximum(even, odd)
    else:
        z = conv
    return jnp.maximum(z + b_ref[...], 0.0).astype(_BF)


def _fused_convs_kernel(x_ref, w1_ref, b1_ref, w2_ref, b2_ref, w3_ref, b3_ref,
                        o_ref, scr1, scr2, *, bblk, seq):
    w1 = w1_ref[...].astype(_BF)
    parts = []
    for b in range(bblk):
        xb = x_ref[b].astype(_BF)
        parts.append(jax.lax.dot_general(
            xb, w1, (((0,), (0,)), ((), ())),
            preferred_element_type=jnp.float32))
    y1 = jnp.concatenate(parts, axis=0) if bblk > 1 else parts[0]
    a1 = _conv_tail(y1, b1_ref, scr1, l_in=seq, pool=True)

    y2 = jnp.dot(a1, w2_ref[...].astype(_BF),
                 preferred_element_type=jnp.float32)
    a2 = _conv_tail(y2, b2_ref, scr2, l_in=seq // 2, pool=True)

    y3 = jnp.dot(a2, w3_ref[...].astype(_BF),
                 preferred_element_type=jnp.float32)
    a3 = _conv_tail(y3, b3_ref, None, l_in=seq // 4, pool=False)

    o_ref[...] = a3


def _fc_head_kernel(a_ref, w1_ref, b1_ref, w2_ref, b2_ref, o_ref, acc_ref):
    @pl.when(pl.program_id(1) == 0)
    def _():
        acc_ref[...] = jnp.zeros_like(acc_ref)

    acc_ref[...] += jnp.dot(a_ref[...], w1_ref[...].astype(_BF),
                            preferred_element_type=jnp.float32)

    @pl.when(pl.program_id(1) == pl.num_programs(1) - 1)
    def _():
        h = jnp.maximum(acc_ref[...] + b1_ref[...], 0.0).astype(_BF)
        out = jnp.dot(h, w2_ref[...].astype(_BF),
                      preferred_element_type=jnp.float32)
        o_ref[...] = out + b2_ref[...]


def kernel(x, c1_w, c1_b, c2_w, c2_b, c3_w, c3_b, f1_w, f1_b, f2_w, f2_b):
    batch, cin, seq = x.shape
    l4 = seq // 4
    bblk = next(d for d in (8, 4, 2, 1) if batch % d == 0)
    m_out = bblk * l4

    body = functools.partial(_fused_convs_kernel, bblk=bblk, seq=seq)
    act = pl.pallas_call(
        body,
        out_shape=jax.ShapeDtypeStruct((batch * l4, 128), _BF),
        grid=(batch // bblk,),
        in_specs=[
            pl.BlockSpec((bblk, cin, seq), lambda i: (i, 0, 0)),
            pl.BlockSpec((cin, 384), lambda i: (0, 0)),
            pl.BlockSpec((1, 128), lambda i: (0, 0)),
            pl.BlockSpec((128, 384), lambda i: (0, 0)),
            pl.BlockSpec((1, 128), lambda i: (0, 0)),
            pl.BlockSpec((128, 384), lambda i: (0, 0)),
            pl.BlockSpec((1, 128), lambda i: (0, 0)),
        ],
        out_specs=pl.BlockSpec((m_out, 128), lambda i: (i, 0)),
        scratch_shapes=[
            pltpu.VMEM((bblk * seq, 128), jnp.float32),
            pltpu.VMEM((bblk * (seq // 2), 128), jnp.float32),
        ],
        compiler_params=pltpu.CompilerParams(
            dimension_semantics=("parallel",)),
    )(x, c1_w, c1_b, c2_w, c2_b, c3_w, c3_b)

    a = act.reshape(batch, l4 * 128)
    k_tot = l4 * 128
    tk = min(8192, k_tot)
    bm = batch // 2 if batch % 2 == 0 else batch
    out = pl.pallas_call(
        _fc_head_kernel,
        out_shape=jax.ShapeDtypeStruct((batch, 128), jnp.float32),
        grid=(batch // bm, k_tot // tk),
        in_specs=[
            pl.BlockSpec((bm, tk), lambda i, ki: (i, ki)),
            pl.BlockSpec((tk, 128), lambda i, ki: (ki, 0)),
            pl.BlockSpec((1, 128), lambda i, ki: (0, 0)),
            pl.BlockSpec((128, 128), lambda i, ki: (0, 0)),
            pl.BlockSpec((1, 128), lambda i, ki: (0, 0)),
        ],
        out_specs=pl.BlockSpec((bm, 128), lambda i, ki: (i, 0)),
        scratch_shapes=[pltpu.VMEM((bm, 128), jnp.float32)],
        compiler_params=pltpu.CompilerParams(
            dimension_semantics=("parallel", "arbitrary")),
    )(a, f1_w, f1_b, f2_w, f2_b)
    return out[:, :10]
